# baseline (speedup 1.0000x reference)
import functools

import jax
import jax.numpy as jnp
from jax import lax
from jax.experimental import pallas as pl
from jax.experimental.pallas import tpu as pltpu
from jax.experimental.pallas import tpu_sc as plsc

N = 10000
N_PAD = 10240
DIN = 128
HID = 128
H = 2
E0 = 320000
BN = 512
NPT = N_PAD // 16
ROWS_PT = 168
EP = ROWS_PT * 16 * 128
NSUP1 = ROWS_PT // 8
ROWS2 = 80
E0P = ROWS2 * 32 * 128

_f32 = jnp.float32
_i32 = jnp.int32


def _tc1_body(x_ref, wg_ref, asrc_ref, adst_ref, ht_ref, aall_ref):
    xb = x_ref[...]
    h = jnp.dot(xb, wg_ref[...], preferred_element_type=_f32)
    h0 = h[:, :HID]
    h1 = h[:, HID:]
    ht_ref[...] = jnp.stack([h0, h1], axis=0)
    a0s = jnp.sum(h0 * asrc_ref[0:1, :], axis=1)
    a1s = jnp.sum(h1 * asrc_ref[1:2, :], axis=1)
    a0d = jnp.sum(h0 * adst_ref[0:1, :], axis=1)
    a1d = jnp.sum(h1 * adst_ref[1:2, :], axis=1)
    z = jnp.zeros((BN,), _f32)
    aall_ref[...] = jnp.stack([a0s, a1s, a0d, a1d, z, z, z, z], axis=0)


_tc1 = pl.pallas_call(
    _tc1_body,
    grid=(N_PAD // BN,),
    in_specs=[
        pl.BlockSpec((BN, DIN), lambda i: (i, 0)),
        pl.BlockSpec((DIN, H * HID), lambda i: (0, 0)),
        pl.BlockSpec((H, HID), lambda i: (0, 0)),
        pl.BlockSpec((H, HID), lambda i: (0, 0)),
    ],
    out_specs=[
        pl.BlockSpec((H, BN, HID), lambda i: (0, i, 0)),
        pl.BlockSpec((8, BN), lambda i: (0, i)),
    ],
    out_shape=[
        jax.ShapeDtypeStruct((H, N_PAD, HID), _f32),
        jax.ShapeDtypeStruct((8, N_PAD), _f32),
    ],
)


def _tc2_body(acc_ref, den_ref, bias_ref, wfc_ref, bfc_ref, st_ref):
    acc = acc_ref[...]
    den = jnp.sum(den_ref[...], axis=1)
    d0 = den[0][:, None] + 1e-16
    d1 = den[1][:, None] + 1e-16
    oc = jnp.concatenate([acc[0] / d0, acc[1] / d1], axis=1)
    oc = oc + bias_ref[...]
    oc = jnp.maximum(oc, 0.01 * oc)
    w = wfc_ref[...]
    dn = (((0,), (1,)), ((), ()))
    ss = lax.dot_general(w[: H * HID], oc, dn, preferred_element_type=_f32)
    sd = lax.dot_general(w[H * HID :], oc, dn, preferred_element_type=_f32)
    sd = sd + bfc_ref[...]
    st_ref[...] = jnp.concatenate([ss, sd, jnp.zeros((2, BN), _f32)], axis=0)


_tc2 = pl.pallas_call(
    _tc2_body,
    grid=(N_PAD // BN,),
    in_specs=[
        pl.BlockSpec((H, BN, HID), lambda i: (0, i, 0)),
        pl.BlockSpec((H, 16, BN), lambda i: (0, 0, i)),
        pl.BlockSpec((1, H * HID), lambda i: (0, 0)),
        pl.BlockSpec((2 * H * HID, 3), lambda i: (0, 0)),
        pl.BlockSpec((3, 1), lambda i: (0, 0)),
    ],
    out_specs=pl.BlockSpec((8, BN), lambda i: (0, i)),
    out_shape=jax.ShapeDtypeStruct((8, N_PAD), _f32),
)

_mesh = plsc.VectorSubcoreMesh(core_axis_name="c", subcore_axis_name="s")


@functools.partial(
    pl.kernel,
    out_type=[
        jax.ShapeDtypeStruct((H, EP), _f32),
        jax.ShapeDtypeStruct((H * 16 * N_PAD,), _f32),
    ],
    mesh=_mesh,
    compiler_params=pltpu.CompilerParams(needs_layout_passes=False),
    scratch_types=[
        pltpu.VMEM((N_PAD,), _f32),
        pltpu.VMEM((N_PAD,), _f32),
        pltpu.VMEM((N_PAD,), _f32),
        pltpu.VMEM((8, 128), _i32),
        pltpu.VMEM((8, 128), _i32),
        pltpu.VMEM((1024,), _f32),
    ],
)
def _sc1a(splain, dplain, aallf, exf, dout, asrc_v, adst_v, denom_v, sidx,
          didx, exv):
    c = lax.axis_index("c")
    s = lax.axis_index("s")
    pltpu.sync_copy(aallf.at[pl.ds(pl.multiple_of(c * N_PAD, 8), N_PAD)], asrc_v)
    pltpu.sync_copy(
        aallf.at[pl.ds(pl.multiple_of((2 + c) * N_PAD, 8), N_PAD)], adst_v
    )

    zeros16 = jnp.zeros((16,), _f32)

    def dzero(i, carry):
        denom_v[pl.ds(i * 16, 16)] = zeros16
        return carry

    lax.fori_loop(0, N_PAD // 16, dzero, 0)

    def sup_body(sb, carry):
        rb = pl.multiple_of(s * ROWS_PT + sb * 8, 8)
        pltpu.sync_copy(splain.at[pl.ds(rb, 8)], sidx)
        pltpu.sync_copy(dplain.at[pl.ds(rb, 8)], didx)
        for r in range(8):
            for g in range(8):
                sl = pl.ds(g * 16, 16)
                di = didx[r, sl]
                av = plsc.load_gather(asrc_v, [sidx[r, sl]]) + plsc.load_gather(
                    adst_v, [di]
                )
                av = jnp.maximum(av, av * 0.2)
                ev = jnp.exp(av)
                exv[pl.ds(r * 128 + g * 16, 16)] = ev
                plsc.addupdate_scatter(denom_v, [di], ev)
        pltpu.sync_copy(
            exv, exf.at[c, pl.ds(pl.multiple_of(rb * 128, 128), 1024)]
        )
        return carry

    lax.fori_loop(0, NSUP1, sup_body, 0)
    pltpu.sync_copy(
        denom_v,
        dout.at[pl.ds(pl.multiple_of((c * 16 + s) * N_PAD, 8), N_PAD)],
    )


@functools.partial(
    pl.kernel,
    out_type=jax.ShapeDtypeStruct((H, N_PAD, HID), _f32),
    mesh=_mesh,
    compiler_params=pltpu.CompilerParams(needs_layout_passes=False),
    scratch_types=[
        pltpu.VMEM((8, 128), _i32),
        pltpu.VMEM((8, 128), _i32),
        pltpu.VMEM((8, 128), _i32),
        pltpu.VMEM((1024,), _f32),
        pltpu.VMEM((2, 128, HID), _f32),
        pltpu.VMEM_SHARED((N_PAD, HID), _f32),
        pltpu.SemaphoreType.DMA,
        pltpu.SemaphoreType.DMA,
    ],
)
def _sc1b(htf, splain, dplain, exf, out, sidx, adjx, didx, exv, rows, acc_sh,
          gsem, ssem):
    c = lax.axis_index("c")
    s = lax.axis_index("s")
    zeros16 = jnp.zeros((16,), _f32)

    def zbody(i, carry):
        for k in range(HID // 16):
            rows[0, i, pl.ds(k * 16, 16)] = zeros16
        return carry

    lax.fori_loop(0, 128, zbody, 0)
    for q in range(NPT // 128):
        pltpu.sync_copy(rows.at[0], acc_sh.at[pl.ds(s * NPT + q * 128, 128)])
    plsc.subcore_barrier()

    coff = c * N_PAD

    def sup_body(sb, carry):
        rb = pl.multiple_of(s * ROWS_PT + sb * 8, 8)
        pltpu.sync_copy(splain.at[pl.ds(rb, 8)], sidx)
        pltpu.sync_copy(dplain.at[pl.ds(rb, 8)], didx)
        pltpu.sync_copy(
            exf.at[c, pl.ds(pl.multiple_of(rb * 128, 128), 1024)], exv
        )
        for r in range(8):
            for g in range(8):
                sl = pl.ds(g * 16, 16)
                adjx[r, sl] = sidx[r, sl] + coff

        gath = [None, None]
        scat = [None, None]
        gath[0] = pltpu.async_copy(htf.at[adjx.at[0]], rows.at[0], gsem)
        for r in range(8):
            b = r % 2
            nb = (r + 1) % 2
            if r < 7:
                if scat[nb] is not None:
                    scat[nb].wait()
                gath[nb] = pltpu.async_copy(
                    htf.at[adjx.at[r + 1]], rows.at[nb], gsem
                )
            gath[b].wait()

            def sbody(g, carry2, r=r, b=b):
                evv = exv[pl.ds(r * 128 + g * 16, 16)]
                for l in range(16):
                    ev = jnp.full((16,), evv[l], _f32)
                    i = g * 16 + l
                    for k in range(HID // 16):
                        sl = pl.ds(k * 16, 16)
                        rows[b, i, sl] = rows[b, i, sl] * ev
                return carry2

            lax.fori_loop(0, 8, sbody, 0, unroll=2)
            scat[b] = pltpu.async_copy(
                rows.at[b], acc_sh.at[didx.at[r]], ssem, add=True
            )
        scat[0].wait()
        scat[1].wait()
        return carry

    lax.fori_loop(0, NSUP1, sup_body, 0)
    plsc.subcore_barrier()
    pltpu.sync_copy(
        acc_sh.at[pl.ds(s * NPT, NPT)], out.at[c, pl.ds(s * NPT, NPT)]
    )


@functools.partial(
    pl.kernel,
    out_type=jax.ShapeDtypeStruct((3, E0P), _f32),
    mesh=_mesh,
    compiler_params=pltpu.CompilerParams(needs_layout_passes=False),
    scratch_types=[
        pltpu.VMEM((6 * N_PAD,), _f32),
        pltpu.VMEM((8, 128), _i32),
        pltpu.VMEM((8, 128), _i32),
        pltpu.VMEM((3, 1024), _f32),
    ],
)
def _sc2(stf, s2d, d2d, out, sv, sidx, didx, stage):
    c = lax.axis_index("c")
    s = lax.axis_index("s")
    wid = s * 2 + c
    pltpu.sync_copy(stf, sv)

    def sup_body(sb, carry):
        rb = pl.multiple_of(wid * ROWS2 + sb * 8, 8)
        pltpu.sync_copy(s2d.at[pl.ds(rb, 8)], sidx)
        pltpu.sync_copy(d2d.at[pl.ds(rb, 8)], didx)
        for r in range(8):
            for g in range(8):
                sl = pl.ds(g * 16, 16)
                si = sidx[r, sl]
                di = didx[r, sl]
                for col in range(3):
                    v = plsc.load_gather(sv, [si + col * N_PAD]) + plsc.load_gather(
                        sv, [di + (3 + col) * N_PAD]
                    )
                    stage[col, pl.ds(r * 128 + g * 16, 16)] = v
        pltpu.sync_copy(
            stage,
            out.at[:, pl.ds(pl.multiple_of(rb * 128, 128), 1024)],
        )
        return carry

    lax.fori_loop(0, ROWS2 // 8, sup_body, 0)


def _tc3_body(in_ref, out_ref):
    b = in_ref[...]
    out_ref[...] = jnp.concatenate(
        [b[0][:, None], b[1][:, None], b[2][:, None]], axis=1
    )


_tc3 = pl.pallas_call(
    _tc3_body,
    grid=(E0 // BN,),
    in_specs=[pl.BlockSpec((3, BN), lambda i: (0, i))],
    out_specs=pl.BlockSpec((BN, 3), lambda i: (i, 0)),
    out_shape=jax.ShapeDtypeStruct((E0, 3), _f32),
)


def kernel(x, edge_index, W_gat, att_src, att_dst, bias_gat, W_fc, b_fc):
    src0 = edge_index[0].astype(_i32)
    dst0 = edge_index[1].astype(_i32)
    xp = jnp.pad(x, ((0, N_PAD - N), (0, 0)))
    ht, aall = _tc1(xp, W_gat, att_src, att_dst)
    htf = ht.reshape(H * N_PAD, HID)
    aallf = aall[:4].reshape(-1)

    loops = jnp.arange(N, dtype=_i32)
    padv = jnp.full((EP - E0 - N,), N, _i32)
    src_all = jnp.concatenate([src0, loops, padv]).reshape(EP // 128, 128)
    dst_all = jnp.concatenate([dst0, loops, padv]).reshape(EP // 128, 128)

    exf, den = _sc1a(src_all, dst_all, aallf)
    acc = _sc1b(htf, src_all, dst_all, exf)
    den16 = den.reshape(H, 16, N_PAD)
    st = _tc2(acc, den16, bias_gat.reshape(1, H * HID), W_fc, b_fc.reshape(3, 1))
    stf = st[:6].reshape(-1)

    epad = jnp.zeros((E0P - E0,), _i32)
    s2d = jnp.concatenate([src0, epad]).reshape(E0P // 128, 128)
    d2d = jnp.concatenate([dst0, epad]).reshape(E0P // 128, 128)
    return _tc3(_sc2(stf, s2d, d2d))


# speedup vs baseline: 1.1778x; 1.1778x over previous
import functools

import jax
import jax.numpy as jnp
from jax import lax
from jax.experimental import pallas as pl
from jax.experimental.pallas import tpu as pltpu
from jax.experimental.pallas import tpu_sc as plsc

N = 10000
N_PAD = 10240
DIN = 128
HID = 128
H = 2
E0 = 320000
BN = 512
NPT = N_PAD // 16
ROWS_PT = 168
EP = ROWS_PT * 16 * 128
NSUP1 = ROWS_PT // 8
ROWS2 = 80
E0P = ROWS2 * 32 * 128

_f32 = jnp.float32
_i32 = jnp.int32


def _tc1_body(x_ref, wg_ref, asrc_ref, adst_ref, ht_ref, aall_ref):
    xb = x_ref[...]
    h = jnp.dot(xb, wg_ref[...], preferred_element_type=_f32)
    h0 = h[:, :HID]
    h1 = h[:, HID:]
    ht_ref[...] = jnp.stack([h0, h1], axis=0)
    a0s = jnp.sum(h0 * asrc_ref[0:1, :], axis=1)
    a1s = jnp.sum(h1 * asrc_ref[1:2, :], axis=1)
    a0d = jnp.sum(h0 * adst_ref[0:1, :], axis=1)
    a1d = jnp.sum(h1 * adst_ref[1:2, :], axis=1)
    z = jnp.zeros((BN,), _f32)
    aall_ref[...] = jnp.stack([a0s, a1s, a0d, a1d, z, z, z, z], axis=0)


_tc1 = pl.pallas_call(
    _tc1_body,
    grid=(N_PAD // BN,),
    in_specs=[
        pl.BlockSpec((BN, DIN), lambda i: (i, 0)),
        pl.BlockSpec((DIN, H * HID), lambda i: (0, 0)),
        pl.BlockSpec((H, HID), lambda i: (0, 0)),
        pl.BlockSpec((H, HID), lambda i: (0, 0)),
    ],
    out_specs=[
        pl.BlockSpec((H, BN, HID), lambda i: (0, i, 0)),
        pl.BlockSpec((8, BN), lambda i: (0, i)),
    ],
    out_shape=[
        jax.ShapeDtypeStruct((H, N_PAD, HID), _f32),
        jax.ShapeDtypeStruct((8, N_PAD), _f32),
    ],
)


def _tc2_body(acc_ref, den_ref, bias_ref, wfc_ref, bfc_ref, st_ref):
    acc = acc_ref[...]
    den = jnp.sum(den_ref[...], axis=1)
    d0 = den[0][:, None] + 1e-16
    d1 = den[1][:, None] + 1e-16
    oc = jnp.concatenate([acc[0] / d0, acc[1] / d1], axis=1)
    oc = oc + bias_ref[...]
    oc = jnp.maximum(oc, 0.01 * oc)
    w = wfc_ref[...]
    dn = (((0,), (1,)), ((), ()))
    ss = lax.dot_general(w[: H * HID], oc, dn, preferred_element_type=_f32)
    sd = lax.dot_general(w[H * HID :], oc, dn, preferred_element_type=_f32)
    sd = sd + bfc_ref[...]
    st_ref[...] = jnp.concatenate([ss, sd, jnp.zeros((2, BN), _f32)], axis=0)


_tc2 = pl.pallas_call(
    _tc2_body,
    grid=(N_PAD // BN,),
    in_specs=[
        pl.BlockSpec((H, BN, HID), lambda i: (0, i, 0)),
        pl.BlockSpec((H, 16, BN), lambda i: (0, 0, i)),
        pl.BlockSpec((1, H * HID), lambda i: (0, 0)),
        pl.BlockSpec((2 * H * HID, 3), lambda i: (0, 0)),
        pl.BlockSpec((3, 1), lambda i: (0, 0)),
    ],
    out_specs=pl.BlockSpec((8, BN), lambda i: (0, i)),
    out_shape=jax.ShapeDtypeStruct((8, N_PAD), _f32),
)

_mesh = plsc.VectorSubcoreMesh(core_axis_name="c", subcore_axis_name="s")


@functools.partial(
    pl.kernel,
    out_type=[
        jax.ShapeDtypeStruct((H, EP), _f32),
        jax.ShapeDtypeStruct((H * 16 * N_PAD,), _f32),
    ],
    mesh=_mesh,
    compiler_params=pltpu.CompilerParams(needs_layout_passes=False),
    scratch_types=[
        pltpu.VMEM((N_PAD,), _f32),
        pltpu.VMEM((N_PAD,), _f32),
        pltpu.VMEM((N_PAD,), _f32),
        pltpu.VMEM((8, 128), _i32),
        pltpu.VMEM((8, 128), _i32),
        pltpu.VMEM((1024,), _f32),
    ],
)
def _sc1a(splain, dplain, aallf, exf, dout, asrc_v, adst_v, denom_v, sidx,
          didx, exv):
    c = lax.axis_index("c")
    s = lax.axis_index("s")
    pltpu.sync_copy(aallf.at[pl.ds(pl.multiple_of(c * N_PAD, 8), N_PAD)], asrc_v)
    pltpu.sync_copy(
        aallf.at[pl.ds(pl.multiple_of((2 + c) * N_PAD, 8), N_PAD)], adst_v
    )

    zeros16 = jnp.zeros((16,), _f32)

    def dzero(i, carry):
        denom_v[pl.ds(i * 16, 16)] = zeros16
        return carry

    lax.fori_loop(0, N_PAD // 16, dzero, 0)

    def sup_body(sb, carry):
        rb = pl.multiple_of(s * ROWS_PT + sb * 8, 8)
        pltpu.sync_copy(splain.at[pl.ds(rb, 8)], sidx)
        pltpu.sync_copy(dplain.at[pl.ds(rb, 8)], didx)
        for r in range(8):
            for g in range(8):
                sl = pl.ds(g * 16, 16)
                di = didx[r, sl]
                av = plsc.load_gather(asrc_v, [sidx[r, sl]]) + plsc.load_gather(
                    adst_v, [di]
                )
                av = jnp.maximum(av, av * 0.2)
                ev = jnp.exp(av)
                exv[pl.ds(r * 128 + g * 16, 16)] = ev
                plsc.addupdate_scatter(denom_v, [di], ev)
        pltpu.sync_copy(
            exv, exf.at[c, pl.ds(pl.multiple_of(rb * 128, 128), 1024)]
        )
        return carry

    lax.fori_loop(0, NSUP1, sup_body, 0)
    pltpu.sync_copy(
        denom_v,
        dout.at[pl.ds(pl.multiple_of((c * 16 + s) * N_PAD, 8), N_PAD)],
    )


@functools.partial(
    pl.kernel,
    out_type=jax.ShapeDtypeStruct((H, N_PAD, HID), _f32),
    mesh=_mesh,
    compiler_params=pltpu.CompilerParams(needs_layout_passes=False),
    scratch_types=[
        pltpu.VMEM((8, 128), _i32),
        pltpu.VMEM((8, 128), _i32),
        pltpu.VMEM((8, 128), _i32),
        pltpu.VMEM((1024,), _f32),
        pltpu.VMEM((2, 128, HID), _f32),
        pltpu.VMEM_SHARED((N_PAD, HID), _f32),
        pltpu.SemaphoreType.DMA,
        pltpu.SemaphoreType.DMA,
    ],
)
def _sc1b(htf, splain, dplain, exf, out, sidx, adjx, didx, exv, rows, acc_sh,
          gsem, ssem):
    c = lax.axis_index("c")
    s = lax.axis_index("s")
    zeros16 = jnp.zeros((16,), _f32)

    def zbody(i, carry):
        for k in range(HID // 16):
            rows[0, i, pl.ds(k * 16, 16)] = zeros16
        return carry

    lax.fori_loop(0, 128, zbody, 0)
    for q in range(NPT // 128):
        pltpu.sync_copy(rows.at[0], acc_sh.at[pl.ds(s * NPT + q * 128, 128)])
    plsc.subcore_barrier()

    coff = c * N_PAD

    def sup_body(sb, carry):
        rb = pl.multiple_of(s * ROWS_PT + sb * 8, 8)
        pltpu.sync_copy(splain.at[pl.ds(rb, 8)], sidx)
        pltpu.sync_copy(dplain.at[pl.ds(rb, 8)], didx)
        pltpu.sync_copy(
            exf.at[c, pl.ds(pl.multiple_of(rb * 128, 128), 1024)], exv
        )
        for r in range(8):
            for g in range(8):
                sl = pl.ds(g * 16, 16)
                adjx[r, sl] = sidx[r, sl] + coff

        gath = [None, None]
        scat = [None, None]
        gath[0] = pltpu.async_copy(htf.at[adjx.at[0]], rows.at[0], gsem)
        for r in range(8):
            b = r % 2
            nb = (r + 1) % 2
            if r < 7:
                if scat[nb] is not None:
                    scat[nb].wait()
                gath[nb] = pltpu.async_copy(
                    htf.at[adjx.at[r + 1]], rows.at[nb], gsem
                )
            gath[b].wait()

            def sbody(g, carry2, r=r, b=b):
                evv = exv[pl.ds(r * 128 + g * 16, 16)]
                for l in range(16):
                    ev = jnp.full((16,), evv[l], _f32)
                    i = g * 16 + l
                    for k in range(HID // 16):
                        sl = pl.ds(k * 16, 16)
                        rows[b, i, sl] = rows[b, i, sl] * ev
                return carry2

            lax.fori_loop(0, 8, sbody, 0, unroll=2)
            scat[b] = pltpu.async_copy(
                rows.at[b], acc_sh.at[didx.at[r]], ssem, add=True
            )
        scat[0].wait()
        scat[1].wait()
        return carry

    lax.fori_loop(0, NSUP1, sup_body, 0)
    plsc.subcore_barrier()
    pltpu.sync_copy(
        acc_sh.at[pl.ds(s * NPT, NPT)], out.at[c, pl.ds(s * NPT, NPT)]
    )


@functools.partial(
    pl.kernel,
    out_type=jax.ShapeDtypeStruct((3, E0P), _f32),
    mesh=_mesh,
    compiler_params=pltpu.CompilerParams(needs_layout_passes=False),
    scratch_types=[
        pltpu.VMEM((6 * N_PAD,), _f32),
        pltpu.VMEM((8, 128), _i32),
        pltpu.VMEM((8, 128), _i32),
        pltpu.VMEM((3, 1024), _f32),
    ],
)
def _sc2(stf, s2d, d2d, out, sv, sidx, didx, stage):
    c = lax.axis_index("c")
    s = lax.axis_index("s")
    wid = s * 2 + c
    pltpu.sync_copy(stf, sv)

    def sup_body(sb, carry):
        rb = pl.multiple_of(wid * ROWS2 + sb * 8, 8)
        pltpu.sync_copy(s2d.at[pl.ds(rb, 8)], sidx)
        pltpu.sync_copy(d2d.at[pl.ds(rb, 8)], didx)
        for r in range(8):
            for g in range(8):
                sl = pl.ds(g * 16, 16)
                si = sidx[r, sl]
                di = didx[r, sl]
                for col in range(3):
                    v = plsc.load_gather(sv, [si + col * N_PAD]) + plsc.load_gather(
                        sv, [di + (3 + col) * N_PAD]
                    )
                    stage[col, pl.ds(r * 128 + g * 16, 16)] = v
        pltpu.sync_copy(
            stage,
            out.at[:, pl.ds(pl.multiple_of(rb * 128, 128), 1024)],
        )
        return carry

    lax.fori_loop(0, ROWS2 // 8, sup_body, 0)


def _tc3_body(in_ref, out_ref):
    b = in_ref[...]
    out_ref[...] = jnp.concatenate(
        [b[0][:, None], b[1][:, None], b[2][:, None]], axis=1
    )


BN3 = 6400

_tc3 = pl.pallas_call(
    _tc3_body,
    grid=(E0 // BN3,),
    in_specs=[pl.BlockSpec((3, BN3), lambda i: (0, i))],
    out_specs=pl.BlockSpec((BN3, 3), lambda i: (i, 0)),
    out_shape=jax.ShapeDtypeStruct((E0, 3), _f32),
)


def kernel(x, edge_index, W_gat, att_src, att_dst, bias_gat, W_fc, b_fc):
    src0 = edge_index[0].astype(_i32)
    dst0 = edge_index[1].astype(_i32)
    xp = jnp.pad(x, ((0, N_PAD - N), (0, 0)))
    ht, aall = _tc1(xp, W_gat, att_src, att_dst)
    htf = ht.reshape(H * N_PAD, HID)
    aallf = aall[:4].reshape(-1)

    loops = jnp.arange(N, dtype=_i32)
    padv = jnp.full((EP - E0 - N,), N, _i32)
    src_all = jnp.concatenate([src0, loops, padv]).reshape(EP // 128, 128)
    dst_all = jnp.concatenate([dst0, loops, padv]).reshape(EP // 128, 128)

    exf, den = _sc1a(src_all, dst_all, aallf)
    acc = _sc1b(htf, src_all, dst_all, exf)
    den16 = den.reshape(H, 16, N_PAD)
    st = _tc2(acc, den16, bias_gat.reshape(1, H * HID), W_fc, b_fc.reshape(3, 1))
    stf = st[:6].reshape(-1)

    epad = jnp.zeros((E0P - E0,), _i32)
    s2d = jnp.concatenate([src0, epad]).reshape(E0P // 128, 128)
    d2d = jnp.concatenate([dst0, epad]).reshape(E0P // 128, 128)
    return _tc3(_sc2(stf, s2d, d2d))


# speedup vs baseline: 1.1910x; 1.0112x over previous
import functools

import jax
import jax.numpy as jnp
from jax import lax
from jax.experimental import pallas as pl
from jax.experimental.pallas import tpu as pltpu
from jax.experimental.pallas import tpu_sc as plsc

N = 10000
N_PAD = 10240
DIN = 128
HID = 128
H = 2
E0 = 320000
BN = 512
NPT = N_PAD // 16
ROWS_PT = 168
EP = ROWS_PT * 16 * 128
NSUP1 = ROWS_PT // 8
ROWS2 = 80
E0P = ROWS2 * 32 * 128

_f32 = jnp.float32
_i32 = jnp.int32


def _tc1_body(x_ref, wg_ref, asrc_ref, adst_ref, ht_ref, aall_ref):
    xb = x_ref[...]
    h = jnp.dot(xb, wg_ref[...], preferred_element_type=_f32)
    h0 = h[:, :HID]
    h1 = h[:, HID:]
    ht_ref[...] = jnp.stack([h0, h1], axis=0)
    a0s = jnp.sum(h0 * asrc_ref[0:1, :], axis=1)
    a1s = jnp.sum(h1 * asrc_ref[1:2, :], axis=1)
    a0d = jnp.sum(h0 * adst_ref[0:1, :], axis=1)
    a1d = jnp.sum(h1 * adst_ref[1:2, :], axis=1)
    z = jnp.zeros((BN,), _f32)
    aall_ref[...] = jnp.stack([a0s, a1s, a0d, a1d, z, z, z, z], axis=0)


_tc1 = pl.pallas_call(
    _tc1_body,
    grid=(N_PAD // BN,),
    in_specs=[
        pl.BlockSpec((BN, DIN), lambda i: (i, 0)),
        pl.BlockSpec((DIN, H * HID), lambda i: (0, 0)),
        pl.BlockSpec((H, HID), lambda i: (0, 0)),
        pl.BlockSpec((H, HID), lambda i: (0, 0)),
    ],
    out_specs=[
        pl.BlockSpec((H, BN, HID), lambda i: (0, i, 0)),
        pl.BlockSpec((8, BN), lambda i: (0, i)),
    ],
    out_shape=[
        jax.ShapeDtypeStruct((H, N_PAD, HID), _f32),
        jax.ShapeDtypeStruct((8, N_PAD), _f32),
    ],
)


def _tc2_body(acc_ref, den_ref, bias_ref, wfc_ref, bfc_ref, st_ref):
    acc = acc_ref[...]
    den = jnp.sum(den_ref[...], axis=1)
    d0 = den[0][:, None] + 1e-16
    d1 = den[1][:, None] + 1e-16
    oc = jnp.concatenate([acc[0] / d0, acc[1] / d1], axis=1)
    oc = oc + bias_ref[...]
    oc = jnp.maximum(oc, 0.01 * oc)
    w = wfc_ref[...]
    dn = (((0,), (1,)), ((), ()))
    ss = lax.dot_general(w[: H * HID], oc, dn, preferred_element_type=_f32)
    sd = lax.dot_general(w[H * HID :], oc, dn, preferred_element_type=_f32)
    sd = sd + bfc_ref[...]
    st_ref[...] = jnp.concatenate([ss, sd, jnp.zeros((2, BN), _f32)], axis=0)


_tc2 = pl.pallas_call(
    _tc2_body,
    grid=(N_PAD // BN,),
    in_specs=[
        pl.BlockSpec((H, BN, HID), lambda i: (0, i, 0)),
        pl.BlockSpec((H, 16, BN), lambda i: (0, 0, i)),
        pl.BlockSpec((1, H * HID), lambda i: (0, 0)),
        pl.BlockSpec((2 * H * HID, 3), lambda i: (0, 0)),
        pl.BlockSpec((3, 1), lambda i: (0, 0)),
    ],
    out_specs=pl.BlockSpec((8, BN), lambda i: (0, i)),
    out_shape=jax.ShapeDtypeStruct((8, N_PAD), _f32),
)

_mesh = plsc.VectorSubcoreMesh(core_axis_name="c", subcore_axis_name="s")


@functools.partial(
    pl.kernel,
    out_type=[
        jax.ShapeDtypeStruct((H, EP), _f32),
        jax.ShapeDtypeStruct((H * 16 * N_PAD,), _f32),
    ],
    mesh=_mesh,
    compiler_params=pltpu.CompilerParams(needs_layout_passes=False),
    scratch_types=[
        pltpu.VMEM((N_PAD,), _f32),
        pltpu.VMEM((N_PAD,), _f32),
        pltpu.VMEM((N_PAD,), _f32),
        pltpu.VMEM((8, 128), _i32),
        pltpu.VMEM((8, 128), _i32),
        pltpu.VMEM((1024,), _f32),
    ],
)
def _sc1a(splain, dplain, aallf, exf, dout, asrc_v, adst_v, denom_v, sidx,
          didx, exv):
    c = lax.axis_index("c")
    s = lax.axis_index("s")
    pltpu.sync_copy(aallf.at[pl.ds(pl.multiple_of(c * N_PAD, 8), N_PAD)], asrc_v)
    pltpu.sync_copy(
        aallf.at[pl.ds(pl.multiple_of((2 + c) * N_PAD, 8), N_PAD)], adst_v
    )

    zeros16 = jnp.zeros((16,), _f32)

    def dzero(i, carry):
        denom_v[pl.ds(i * 16, 16)] = zeros16
        return carry

    lax.fori_loop(0, N_PAD // 16, dzero, 0)

    def sup_body(sb, carry):
        rb = pl.multiple_of(s * ROWS_PT + sb * 8, 8)
        pltpu.sync_copy(splain.at[pl.ds(rb, 8)], sidx)
        pltpu.sync_copy(dplain.at[pl.ds(rb, 8)], didx)
        for r in range(8):
            for g in range(8):
                sl = pl.ds(g * 16, 16)
                di = didx[r, sl]
                av = plsc.load_gather(asrc_v, [sidx[r, sl]]) + plsc.load_gather(
                    adst_v, [di]
                )
                av = jnp.maximum(av, av * 0.2)
                ev = jnp.exp(av)
                exv[pl.ds(r * 128 + g * 16, 16)] = ev
                plsc.addupdate_scatter(denom_v, [di], ev)
        pltpu.sync_copy(
            exv, exf.at[c, pl.ds(pl.multiple_of(rb * 128, 128), 1024)]
        )
        return carry

    lax.fori_loop(0, NSUP1, sup_body, 0)
    pltpu.sync_copy(
        denom_v,
        dout.at[pl.ds(pl.multiple_of((c * 16 + s) * N_PAD, 8), N_PAD)],
    )


@functools.partial(
    pl.kernel,
    out_type=jax.ShapeDtypeStruct((H, N_PAD, HID), _f32),
    mesh=_mesh,
    compiler_params=pltpu.CompilerParams(needs_layout_passes=False),
    scratch_types=[
        pltpu.VMEM((8, 128), _i32),
        pltpu.VMEM((8, 128), _i32),
        pltpu.VMEM((8, 128), _i32),
        pltpu.VMEM((1024,), _f32),
        pltpu.VMEM((2, 128, HID), _f32),
        pltpu.VMEM_SHARED((N_PAD, HID), _f32),
        pltpu.SemaphoreType.DMA,
        pltpu.SemaphoreType.DMA,
        pltpu.SemaphoreType.DMA,
    ],
)
def _sc1b(htf, splain, dplain, exf, out, sidx, adjx, didx, exv, rows, acc_sh,
          gsem, ssem, isem):
    c = lax.axis_index("c")
    s = lax.axis_index("s")
    zeros16 = jnp.zeros((16,), _f32)

    def zbody(i, carry):
        for k in range(HID // 16):
            rows[0, i, pl.ds(k * 16, 16)] = zeros16
        return carry

    lax.fori_loop(0, 128, zbody, 0)
    for q in range(NPT // 128):
        pltpu.sync_copy(rows.at[0], acc_sh.at[pl.ds(s * NPT + q * 128, 128)])
    plsc.subcore_barrier()

    coff = c * N_PAD

    def sup_body(sb, carry):
        rb = pl.multiple_of(s * ROWS_PT + sb * 8, 8)
        icps = [
            pltpu.async_copy(splain.at[pl.ds(rb, 8)], sidx, isem),
            pltpu.async_copy(dplain.at[pl.ds(rb, 8)], didx, isem),
            pltpu.async_copy(
                exf.at[c, pl.ds(pl.multiple_of(rb * 128, 128), 1024)], exv, isem
            ),
        ]
        for cp in icps:
            cp.wait()
        for r in range(8):
            for g in range(8):
                sl = pl.ds(g * 16, 16)
                adjx[r, sl] = sidx[r, sl] + coff

        gath = [None, None]
        scat = [None, None]
        gath[0] = pltpu.async_copy(htf.at[adjx.at[0]], rows.at[0], gsem)
        for r in range(8):
            b = r % 2
            nb = (r + 1) % 2
            if r < 7:
                if scat[nb] is not None:
                    scat[nb].wait()
                gath[nb] = pltpu.async_copy(
                    htf.at[adjx.at[r + 1]], rows.at[nb], gsem
                )
            gath[b].wait()

            def sbody(g, carry2, r=r, b=b):
                evv = exv[pl.ds(r * 128 + g * 16, 16)]
                for l in range(16):
                    ev = jnp.full((16,), evv[l], _f32)
                    i = g * 16 + l
                    for k in range(HID // 16):
                        sl = pl.ds(k * 16, 16)
                        rows[b, i, sl] = rows[b, i, sl] * ev
                return carry2

            lax.fori_loop(0, 8, sbody, 0, unroll=2)
            scat[b] = pltpu.async_copy(
                rows.at[b], acc_sh.at[didx.at[r]], ssem, add=True
            )
        scat[0].wait()
        scat[1].wait()
        return carry

    lax.fori_loop(0, NSUP1, sup_body, 0)
    plsc.subcore_barrier()
    pltpu.sync_copy(
        acc_sh.at[pl.ds(s * NPT, NPT)], out.at[c, pl.ds(s * NPT, NPT)]
    )


@functools.partial(
    pl.kernel,
    out_type=jax.ShapeDtypeStruct((3, E0P), _f32),
    mesh=_mesh,
    compiler_params=pltpu.CompilerParams(needs_layout_passes=False),
    scratch_types=[
        pltpu.VMEM((6 * N_PAD,), _f32),
        pltpu.VMEM((8, 128), _i32),
        pltpu.VMEM((8, 128), _i32),
        pltpu.VMEM((3, 1024), _f32),
    ],
)
def _sc2(stf, s2d, d2d, out, sv, sidx, didx, stage):
    c = lax.axis_index("c")
    s = lax.axis_index("s")
    wid = s * 2 + c
    pltpu.sync_copy(stf, sv)

    def sup_body(sb, carry):
        rb = pl.multiple_of(wid * ROWS2 + sb * 8, 8)
        pltpu.sync_copy(s2d.at[pl.ds(rb, 8)], sidx)
        pltpu.sync_copy(d2d.at[pl.ds(rb, 8)], didx)
        for r in range(8):
            for g in range(8):
                sl = pl.ds(g * 16, 16)
                si = sidx[r, sl]
                di = didx[r, sl]
                for col in range(3):
                    v = plsc.load_gather(sv, [si + col * N_PAD]) + plsc.load_gather(
                        sv, [di + (3 + col) * N_PAD]
                    )
                    stage[col, pl.ds(r * 128 + g * 16, 16)] = v
        pltpu.sync_copy(
            stage,
            out.at[:, pl.ds(pl.multiple_of(rb * 128, 128), 1024)],
        )
        return carry

    lax.fori_loop(0, ROWS2 // 8, sup_body, 0)


def _tc3_body(in_ref, out_ref):
    b = in_ref[...]
    out_ref[...] = jnp.concatenate(
        [b[0][:, None], b[1][:, None], b[2][:, None]], axis=1
    )


BN3 = 6400

_tc3 = pl.pallas_call(
    _tc3_body,
    grid=(E0 // BN3,),
    in_specs=[pl.BlockSpec((3, BN3), lambda i: (0, i))],
    out_specs=pl.BlockSpec((BN3, 3), lambda i: (i, 0)),
    out_shape=jax.ShapeDtypeStruct((E0, 3), _f32),
)


def kernel(x, edge_index, W_gat, att_src, att_dst, bias_gat, W_fc, b_fc):
    src0 = edge_index[0].astype(_i32)
    dst0 = edge_index[1].astype(_i32)
    xp = jnp.pad(x, ((0, N_PAD - N), (0, 0)))
    ht, aall = _tc1(xp, W_gat, att_src, att_dst)
    htf = ht.reshape(H * N_PAD, HID)
    aallf = aall[:4].reshape(-1)

    loops = jnp.arange(N, dtype=_i32)
    padv = jnp.full((EP - E0 - N,), N, _i32)
    src_all = jnp.concatenate([src0, loops, padv]).reshape(EP // 128, 128)
    dst_all = jnp.concatenate([dst0, loops, padv]).reshape(EP // 128, 128)

    exf, den = _sc1a(src_all, dst_all, aallf)
    acc = _sc1b(htf, src_all, dst_all, exf)
    den16 = den.reshape(H, 16, N_PAD)
    st = _tc2(acc, den16, bias_gat.reshape(1, H * HID), W_fc, b_fc.reshape(3, 1))
    stf = st[:6].reshape(-1)

    epad = jnp.zeros((E0P - E0,), _i32)
    s2d = jnp.concatenate([src0, epad]).reshape(E0P // 128, 128)
    d2d = jnp.concatenate([dst0, epad]).reshape(E0P // 128, 128)
    return _tc3(_sc2(stf, s2d, d2d))


# speedup vs baseline: 1.2045x; 1.0114x over previous
import functools

import jax
import jax.numpy as jnp
from jax import lax
from jax.experimental import pallas as pl
from jax.experimental.pallas import tpu as pltpu
from jax.experimental.pallas import tpu_sc as plsc

N = 10000
N_PAD = 10240
DIN = 128
HID = 128
H = 2
E0 = 320000
BN = 512
NPT = N_PAD // 16
ROWS_PT = 168
EP = ROWS_PT * 16 * 128
NSUP1 = ROWS_PT // 8
ROWS2 = 80
E0P = ROWS2 * 32 * 128

_f32 = jnp.float32
_i32 = jnp.int32


def _tc1_body(x_ref, wg_ref, asrc_ref, adst_ref, ht_ref, aall_ref):
    xb = x_ref[...]
    h = jnp.dot(xb, wg_ref[...], preferred_element_type=_f32)
    h0 = h[:, :HID]
    h1 = h[:, HID:]
    ht_ref[...] = jnp.stack([h0, h1], axis=0)
    a0s = jnp.sum(h0 * asrc_ref[0:1, :], axis=1)
    a1s = jnp.sum(h1 * asrc_ref[1:2, :], axis=1)
    a0d = jnp.sum(h0 * adst_ref[0:1, :], axis=1)
    a1d = jnp.sum(h1 * adst_ref[1:2, :], axis=1)
    z = jnp.zeros((BN,), _f32)
    aall_ref[...] = jnp.stack([a0s, a1s, a0d, a1d, z, z, z, z], axis=0)


_tc1 = pl.pallas_call(
    _tc1_body,
    grid=(N_PAD // BN,),
    in_specs=[
        pl.BlockSpec((BN, DIN), lambda i: (i, 0)),
        pl.BlockSpec((DIN, H * HID), lambda i: (0, 0)),
        pl.BlockSpec((H, HID), lambda i: (0, 0)),
        pl.BlockSpec((H, HID), lambda i: (0, 0)),
    ],
    out_specs=[
        pl.BlockSpec((H, BN, HID), lambda i: (0, i, 0)),
        pl.BlockSpec((8, BN), lambda i: (0, i)),
    ],
    out_shape=[
        jax.ShapeDtypeStruct((H, N_PAD, HID), _f32),
        jax.ShapeDtypeStruct((8, N_PAD), _f32),
    ],
)


def _tc2_body(acc_ref, den_ref, bias_ref, wfc_ref, bfc_ref, st_ref):
    acc = acc_ref[...]
    den = jnp.sum(den_ref[...], axis=1)
    d0 = den[0][:, None] + 1e-16
    d1 = den[1][:, None] + 1e-16
    oc = jnp.concatenate([acc[0] / d0, acc[1] / d1], axis=1)
    oc = oc + bias_ref[...]
    oc = jnp.maximum(oc, 0.01 * oc)
    w = wfc_ref[...]
    dn = (((0,), (1,)), ((), ()))
    ss = lax.dot_general(w[: H * HID], oc, dn, preferred_element_type=_f32)
    sd = lax.dot_general(w[H * HID :], oc, dn, preferred_element_type=_f32)
    sd = sd + bfc_ref[...]
    st_ref[...] = jnp.concatenate([ss, sd, jnp.zeros((2, BN), _f32)], axis=0)


_tc2 = pl.pallas_call(
    _tc2_body,
    grid=(N_PAD // BN,),
    in_specs=[
        pl.BlockSpec((H, BN, HID), lambda i: (0, i, 0)),
        pl.BlockSpec((H, 16, BN), lambda i: (0, 0, i)),
        pl.BlockSpec((1, H * HID), lambda i: (0, 0)),
        pl.BlockSpec((2 * H * HID, 3), lambda i: (0, 0)),
        pl.BlockSpec((3, 1), lambda i: (0, 0)),
    ],
    out_specs=pl.BlockSpec((8, BN), lambda i: (0, i)),
    out_shape=jax.ShapeDtypeStruct((8, N_PAD), _f32),
)

_mesh = plsc.VectorSubcoreMesh(core_axis_name="c", subcore_axis_name="s")


@functools.partial(
    pl.kernel,
    out_type=[
        jax.ShapeDtypeStruct((H, EP), _f32),
        jax.ShapeDtypeStruct((H * 16 * N_PAD,), _f32),
    ],
    mesh=_mesh,
    compiler_params=pltpu.CompilerParams(needs_layout_passes=False),
    scratch_types=[
        pltpu.VMEM((N_PAD,), _f32),
        pltpu.VMEM((N_PAD,), _f32),
        pltpu.VMEM((N_PAD,), _f32),
        pltpu.VMEM((8, 128), _i32),
        pltpu.VMEM((8, 128), _i32),
        pltpu.VMEM((1024,), _f32),
        pltpu.SemaphoreType.DMA,
    ],
)
def _sc1a(splain, dplain, aallf, exf, dout, asrc_v, adst_v, denom_v, sidx,
          didx, exv, isem):
    c = lax.axis_index("c")
    s = lax.axis_index("s")
    pltpu.sync_copy(aallf.at[pl.ds(pl.multiple_of(c * N_PAD, 8), N_PAD)], asrc_v)
    pltpu.sync_copy(
        aallf.at[pl.ds(pl.multiple_of((2 + c) * N_PAD, 8), N_PAD)], adst_v
    )

    zeros16 = jnp.zeros((16,), _f32)

    def dzero(i, carry):
        denom_v[pl.ds(i * 16, 16)] = zeros16
        return carry

    lax.fori_loop(0, N_PAD // 16, dzero, 0)

    def sup_body(sb, carry):
        rb = pl.multiple_of(s * ROWS_PT + sb * 8, 8)
        icps = [
            pltpu.async_copy(splain.at[pl.ds(rb, 8)], sidx, isem),
            pltpu.async_copy(dplain.at[pl.ds(rb, 8)], didx, isem),
        ]
        for cp in icps:
            cp.wait()
        for r in range(8):
            for g in range(8):
                sl = pl.ds(g * 16, 16)
                di = didx[r, sl]
                av = plsc.load_gather(asrc_v, [sidx[r, sl]]) + plsc.load_gather(
                    adst_v, [di]
                )
                av = jnp.maximum(av, av * 0.2)
                ev = jnp.exp(av)
                exv[pl.ds(r * 128 + g * 16, 16)] = ev
                plsc.addupdate_scatter(denom_v, [di], ev)
        pltpu.sync_copy(
            exv, exf.at[c, pl.ds(pl.multiple_of(rb * 128, 128), 1024)]
        )
        return carry

    lax.fori_loop(0, NSUP1, sup_body, 0)
    pltpu.sync_copy(
        denom_v,
        dout.at[pl.ds(pl.multiple_of((c * 16 + s) * N_PAD, 8), N_PAD)],
    )


@functools.partial(
    pl.kernel,
    out_type=jax.ShapeDtypeStruct((H, N_PAD, HID), _f32),
    mesh=_mesh,
    compiler_params=pltpu.CompilerParams(needs_layout_passes=False),
    scratch_types=[
        pltpu.VMEM((8, 128), _i32),
        pltpu.VMEM((8, 128), _i32),
        pltpu.VMEM((8, 128), _i32),
        pltpu.VMEM((1024,), _f32),
        pltpu.VMEM((2, 128, HID), _f32),
        pltpu.VMEM_SHARED((N_PAD, HID), _f32),
        pltpu.SemaphoreType.DMA,
        pltpu.SemaphoreType.DMA,
        pltpu.SemaphoreType.DMA,
    ],
)
def _sc1b(htf, splain, dplain, exf, out, sidx, adjx, didx, exv, rows, acc_sh,
          gsem, ssem, isem):
    c = lax.axis_index("c")
    s = lax.axis_index("s")
    zeros16 = jnp.zeros((16,), _f32)

    def zbody(i, carry):
        for k in range(HID // 16):
            rows[0, i, pl.ds(k * 16, 16)] = zeros16
        return carry

    lax.fori_loop(0, 128, zbody, 0)
    for q in range(NPT // 128):
        pltpu.sync_copy(rows.at[0], acc_sh.at[pl.ds(s * NPT + q * 128, 128)])
    plsc.subcore_barrier()

    coff = c * N_PAD

    def sup_body(sb, carry):
        rb = pl.multiple_of(s * ROWS_PT + sb * 8, 8)
        icps = [
            pltpu.async_copy(splain.at[pl.ds(rb, 8)], sidx, isem),
            pltpu.async_copy(dplain.at[pl.ds(rb, 8)], didx, isem),
            pltpu.async_copy(
                exf.at[c, pl.ds(pl.multiple_of(rb * 128, 128), 1024)], exv, isem
            ),
        ]
        for cp in icps:
            cp.wait()
        for r in range(8):
            for g in range(8):
                sl = pl.ds(g * 16, 16)
                adjx[r, sl] = sidx[r, sl] + coff

        gath = [None, None]
        scat = [None, None]
        gath[0] = pltpu.async_copy(htf.at[adjx.at[0]], rows.at[0], gsem)
        for r in range(8):
            b = r % 2
            nb = (r + 1) % 2
            if r < 7:
                if scat[nb] is not None:
                    scat[nb].wait()
                gath[nb] = pltpu.async_copy(
                    htf.at[adjx.at[r + 1]], rows.at[nb], gsem
                )
            gath[b].wait()

            def sbody(g, carry2, r=r, b=b):
                evv = exv[pl.ds(r * 128 + g * 16, 16)]
                for l in range(16):
                    ev = jnp.full((16,), evv[l], _f32)
                    i = g * 16 + l
                    for k in range(HID // 16):
                        sl = pl.ds(k * 16, 16)
                        rows[b, i, sl] = rows[b, i, sl] * ev
                return carry2

            lax.fori_loop(0, 8, sbody, 0, unroll=2)
            scat[b] = pltpu.async_copy(
                rows.at[b], acc_sh.at[didx.at[r]], ssem, add=True
            )
        scat[0].wait()
        scat[1].wait()
        return carry

    lax.fori_loop(0, NSUP1, sup_body, 0)
    plsc.subcore_barrier()
    pltpu.sync_copy(
        acc_sh.at[pl.ds(s * NPT, NPT)], out.at[c, pl.ds(s * NPT, NPT)]
    )


@functools.partial(
    pl.kernel,
    out_type=jax.ShapeDtypeStruct((3, E0P), _f32),
    mesh=_mesh,
    compiler_params=pltpu.CompilerParams(needs_layout_passes=False),
    scratch_types=[
        pltpu.VMEM((6 * N_PAD,), _f32),
        pltpu.VMEM((8, 128), _i32),
        pltpu.VMEM((8, 128), _i32),
        pltpu.VMEM((3, 1024), _f32),
        pltpu.SemaphoreType.DMA,
    ],
)
def _sc2(stf, s2d, d2d, out, sv, sidx, didx, stage, isem):
    c = lax.axis_index("c")
    s = lax.axis_index("s")
    wid = s * 2 + c
    pltpu.sync_copy(stf, sv)

    def sup_body(sb, carry):
        rb = pl.multiple_of(wid * ROWS2 + sb * 8, 8)
        icps = [
            pltpu.async_copy(s2d.at[pl.ds(rb, 8)], sidx, isem),
            pltpu.async_copy(d2d.at[pl.ds(rb, 8)], didx, isem),
        ]
        for cp in icps:
            cp.wait()
        for r in range(8):
            for g in range(8):
                sl = pl.ds(g * 16, 16)
                si = sidx[r, sl]
                di = didx[r, sl]
                for col in range(3):
                    v = plsc.load_gather(sv, [si + col * N_PAD]) + plsc.load_gather(
                        sv, [di + (3 + col) * N_PAD]
                    )
                    stage[col, pl.ds(r * 128 + g * 16, 16)] = v
        pltpu.sync_copy(
            stage,
            out.at[:, pl.ds(pl.multiple_of(rb * 128, 128), 1024)],
        )
        return carry

    lax.fori_loop(0, ROWS2 // 8, sup_body, 0)


def _tc3_body(in_ref, out_ref):
    b = in_ref[...]
    out_ref[...] = jnp.concatenate(
        [b[0][:, None], b[1][:, None], b[2][:, None]], axis=1
    )


BN3 = 6400

_tc3 = pl.pallas_call(
    _tc3_body,
    grid=(E0 // BN3,),
    in_specs=[pl.BlockSpec((3, BN3), lambda i: (0, i))],
    out_specs=pl.BlockSpec((BN3, 3), lambda i: (i, 0)),
    out_shape=jax.ShapeDtypeStruct((E0, 3), _f32),
)


def kernel(x, edge_index, W_gat, att_src, att_dst, bias_gat, W_fc, b_fc):
    src0 = edge_index[0].astype(_i32)
    dst0 = edge_index[1].astype(_i32)
    xp = jnp.pad(x, ((0, N_PAD - N), (0, 0)))
    ht, aall = _tc1(xp, W_gat, att_src, att_dst)
    htf = ht.reshape(H * N_PAD, HID)
    aallf = aall[:4].reshape(-1)

    loops = jnp.arange(N, dtype=_i32)
    padv = jnp.full((EP - E0 - N,), N, _i32)
    src_all = jnp.concatenate([src0, loops, padv]).reshape(EP // 128, 128)
    dst_all = jnp.concatenate([dst0, loops, padv]).reshape(EP // 128, 128)

    exf, den = _sc1a(src_all, dst_all, aallf)
    acc = _sc1b(htf, src_all, dst_all, exf)
    den16 = den.reshape(H, 16, N_PAD)
    st = _tc2(acc, den16, bias_gat.reshape(1, H * HID), W_fc, b_fc.reshape(3, 1))
    stf = st[:6].reshape(-1)

    epad = jnp.zeros((E0P - E0,), _i32)
    s2d = jnp.concatenate([src0, epad]).reshape(E0P // 128, 128)
    d2d = jnp.concatenate([dst0, epad]).reshape(E0P // 128, 128)
    return _tc3(_sc2(stf, s2d, d2d))
